# baseline (speedup 1.0000x reference)
import functools

import jax
import jax.numpy as jnp
from jax import lax
from jax.experimental import pallas as pl
from jax.experimental.pallas import tpu as pltpu
from jax.experimental.pallas import tpu_sc as plsc

NC, NS, L = 2, 16, 16
NW = NC * NS
T = 32


def _sc_call(N, H, NBINS):
    TPW = N // NW
    CHUNKS = TPW // T
    CH = H // L
    JV = TPW // L

    mesh = plsc.VectorSubcoreMesh(core_axis_name="c", subcore_axis_name="s")

    @functools.partial(
        pl.kernel,
        out_type=jax.ShapeDtypeStruct((N, H), jnp.float32),
        mesh=mesh,
        compiler_params=pltpu.CompilerParams(needs_layout_passes=False),
        scratch_types=[
            pltpu.VMEM((2, T, H), jnp.float32),
            pltpu.VMEM((2, T, H), jnp.float32),
            pltpu.VMEM((2, T, H), jnp.float32),
            pltpu.VMEM((TPW,), jnp.float32),
            pltpu.VMEM((TPW,), jnp.float32),
            pltpu.VMEM((TPW,), jnp.int32),
            pltpu.VMEM((TPW,), jnp.int32),
            pltpu.VMEM((2 * NBINS,), jnp.float32),
            pltpu.SemaphoreType.DMA,
            pltpu.SemaphoreType.DMA,
            pltpu.SemaphoreType.DMA,
            pltpu.SemaphoreType.DMA,
            pltpu.SemaphoreType.DMA,
            pltpu.SemaphoreType.DMA,
            pltpu.SemaphoreType.DMA,
            pltpu.SemaphoreType.DMA,
        ],
    )
    def body(enc_hbm, pv_hbm, ev_hbm, ctab_hbm, bnd_hbm, out_hbm,
             out_b, prow_b, erow_b, pvals, evals, pidx, eidx, bnds,
             se0, se1, sp0, sp1, sg0, sg1, sw0, sw1):
        wid = lax.axis_index("s") * NC + lax.axis_index("c")
        wbase = wid * TPW
        se, sp, sg, sw = (se0, se1), (sp0, sp1), (sg0, sg1), (sw0, sw1)

        pltpu.sync_copy(bnd_hbm, bnds)
        pltpu.sync_copy(pv_hbm.at[pl.ds(wbase, TPW)], pvals)
        pltpu.sync_copy(ev_hbm.at[pl.ds(wbase, TPW)], evals)

        @pl.loop(0, JV)
        def _search(j):
            sl = pl.ds(j * L, L)
            for vals_ref, idx_ref, base_bin in ((pvals, pidx, 0),
                                                (evals, eidx, NBINS)):
                v = vals_ref[sl]
                curr = jnp.zeros((L,), jnp.int32)
                step = NBINS // 2
                while step >= 1:
                    probe = plsc.load_gather(bnds, [curr + (base_bin + step - 1)])
                    curr = jnp.where(probe < v, curr + step, curr)
                    step //= 2
                idx_ref[sl] = curr + base_bin

        def issue(c, p):
            base = wbase + c * T
            pltpu.async_copy(enc_hbm.at[pl.ds(base, T)], out_b.at[p], se[p])
            pltpu.async_copy(ctab_hbm.at[pidx.at[pl.ds(c * T, T)]],
                             prow_b.at[p], sp[p])
            pltpu.async_copy(ctab_hbm.at[eidx.at[pl.ds(c * T, T)]],
                             erow_b.at[p], sg[p])

        def wait_wb(p):
            pltpu.make_async_copy(out_b.at[p], out_hbm.at[pl.ds(wbase, T)],
                                  sw[p]).wait()

        def finish(c, p):
            base = wbase + c * T
            pltpu.make_async_copy(enc_hbm.at[pl.ds(base, T)], out_b.at[p],
                                  se[p]).wait()
            pltpu.make_async_copy(ctab_hbm.at[pidx.at[pl.ds(c * T, T)]],
                                  prow_b.at[p], sp[p]).wait()
            pltpu.make_async_copy(ctab_hbm.at[eidx.at[pl.ds(c * T, T)]],
                                  erow_b.at[p], sg[p]).wait()

            @pl.loop(0, T)
            def _row(t):
                for h in range(CH):
                    sl = pl.ds(h * L, L)
                    plsc.addupdate(out_b.at[p, t, sl],
                                   prow_b[p, t, sl] + erow_b[p, t, sl])

            pltpu.async_copy(out_b.at[p], out_hbm.at[pl.ds(base, T)], sw[p])

        issue(0, 0)

        @pl.loop(0, CHUNKS, step=2)
        def _main(cc):
            @pl.when(cc > 0)
            def _():
                wait_wb(1)

            issue(cc + 1, 1)
            finish(cc, 0)

            @pl.when(cc + 2 < CHUNKS)
            def _():
                wait_wb(0)
                issue(cc + 2, 0)

            finish(cc + 1, 1)

        wait_wb(0)
        wait_wb(1)

    return body


def kernel(encoder_output, pitch_target, energy_target, pitch_table, energy_table):
    B, S, H = encoder_output.shape
    N = B * S
    NBINS = pitch_table.shape[0]
    enc = encoder_output.reshape(N, H)
    pv = pitch_target.reshape(N)
    ev = energy_target.reshape(N)
    ctab = jnp.concatenate([pitch_table, energy_table], axis=0)
    bnds = jnp.concatenate([
        jnp.linspace(50.0, 400.0, NBINS),
        jnp.linspace(0.0, 1.0, NBINS),
    ])
    out = _sc_call(N, H, NBINS)(enc, pv, ev, ctab, bnds)
    return out.reshape(B, S, H)


# speedup vs baseline: 6.0199x; 6.0199x over previous
import functools

import jax
import jax.numpy as jnp
from jax import lax
from jax.experimental import pallas as pl
from jax.experimental.pallas import tpu as pltpu
from jax.experimental.pallas import tpu_sc as plsc

NC, NS, L = 2, 16, 16
NW = NC * NS
CW = 128
T = 128


def _sc_call(N, H, NBINS):
    SPT = N // NS
    CG = H // CW
    TS = NW // CG
    NPS = N // TS
    CHUNKS = NPS // T
    R = 2 * NBINS
    CH = CW // L

    mesh = plsc.VectorSubcoreMesh(core_axis_name="c", subcore_axis_name="s")

    @functools.partial(
        pl.kernel,
        out_type=jax.ShapeDtypeStruct((N, H), jnp.float32),
        mesh=mesh,
        compiler_params=pltpu.CompilerParams(needs_layout_passes=False),
        scratch_types=[
            pltpu.VMEM((R, CW), jnp.float32),
            pltpu.VMEM((2, T, CW), jnp.float32),
            pltpu.VMEM((2, T), jnp.int32),
            pltpu.VMEM((2, T), jnp.int32),
            pltpu.VMEM((SPT,), jnp.float32),
            pltpu.VMEM((SPT,), jnp.float32),
            pltpu.VMEM((SPT,), jnp.int32),
            pltpu.VMEM((SPT,), jnp.int32),
            pltpu.VMEM((2 * NBINS,), jnp.float32),
            pltpu.VMEM_SHARED((N,), jnp.int32),
            pltpu.VMEM_SHARED((N,), jnp.int32),
            pltpu.SemaphoreType.DMA,
            pltpu.SemaphoreType.DMA,
            pltpu.SemaphoreType.DMA,
            pltpu.SemaphoreType.DMA,
            pltpu.SemaphoreType.DMA,
            pltpu.SemaphoreType.DMA,
        ],
    )
    def body(enc_hbm, pv_hbm, ev_hbm, ctab_hbm, bnd_hbm, out_hbm,
             tab, out_b, pb_b, eb_b, pvals, evals, pidx, eidx, bnds,
             pidx_sh, eidx_sh,
             se0, se1, si0, si1, sw0, sw1):
        cid = lax.axis_index("c")
        sid = lax.axis_index("s")
        wid = cid * NS + sid
        gcol = (wid % CG) * CW
        tok0 = (wid // CG) * NPS
        se, si, sw = (se0, se1), (si0, si1), (sw0, sw1)

        pltpu.sync_copy(bnd_hbm, bnds)
        qbase = sid * SPT
        pltpu.sync_copy(pv_hbm.at[pl.ds(qbase, SPT)], pvals)
        pltpu.sync_copy(ev_hbm.at[pl.ds(qbase, SPT)], evals)

        @pl.loop(0, SPT // L)
        def _search(j):
            sl = pl.ds(j * L, L)
            for vals_ref, idx_ref, base_bin in ((pvals, pidx, 0),
                                                (evals, eidx, NBINS)):
                v = vals_ref[sl]
                curr = jnp.zeros((L,), jnp.int32)
                step = NBINS // 2
                while step >= 1:
                    probe = plsc.load_gather(bnds, [curr + (base_bin + step - 1)])
                    curr = jnp.where(probe < v, curr + step, curr)
                    step //= 2
                idx_ref[sl] = curr + base_bin

        pltpu.sync_copy(pidx, pidx_sh.at[pl.ds(qbase, SPT)])
        pltpu.sync_copy(eidx, eidx_sh.at[pl.ds(qbase, SPT)])

        pltpu.sync_copy(ctab_hbm.at[:, pl.ds(gcol, CW)], tab)
        plsc.subcore_barrier()

        def issue(c, p):
            base = tok0 + c * T
            pltpu.async_copy(enc_hbm.at[pl.ds(base, T), pl.ds(gcol, CW)],
                             out_b.at[p], se[p])
            pltpu.async_copy(pidx_sh.at[pl.ds(base, T)], pb_b.at[p], si[p])
            pltpu.async_copy(eidx_sh.at[pl.ds(base, T)], eb_b.at[p], si[p])

        def wait_wb(p):
            pltpu.make_async_copy(out_b.at[p],
                                  out_hbm.at[pl.ds(tok0, T), pl.ds(gcol, CW)],
                                  sw[p]).wait()

        def finish(c, p):
            base = tok0 + c * T
            pltpu.make_async_copy(enc_hbm.at[pl.ds(base, T), pl.ds(gcol, CW)],
                                  out_b.at[p], se[p]).wait()
            pltpu.make_async_copy(pidx_sh.at[pl.ds(base, T)], pb_b.at[p],
                                  si[p]).wait()
            pltpu.make_async_copy(eidx_sh.at[pl.ds(base, T)], eb_b.at[p],
                                  si[p]).wait()

            @pl.loop(0, T // L)
            def _row(j):
                t0 = j * L
                rpv = pb_b[p, pl.ds(t0, L)]
                rev = eb_b[p, pl.ds(t0, L)]
                for k in range(L):
                    for h in range(CH):
                        sl = pl.ds(h * L, L)
                        plsc.addupdate(out_b.at[p, t0 + k, sl],
                                       tab[rpv[k], sl] + tab[rev[k], sl])

            pltpu.async_copy(out_b.at[p],
                             out_hbm.at[pl.ds(base, T), pl.ds(gcol, CW)],
                             sw[p])

        issue(0, 0)

        @pl.loop(0, CHUNKS, step=2)
        def _main(cc):
            @pl.when(cc > 0)
            def _():
                wait_wb(1)

            issue(cc + 1, 1)
            finish(cc, 0)

            @pl.when(cc + 2 < CHUNKS)
            def _():
                wait_wb(0)
                issue(cc + 2, 0)

            finish(cc + 1, 1)

        wait_wb(0)
        wait_wb(1)

    return body


def kernel(encoder_output, pitch_target, energy_target, pitch_table, energy_table):
    B, S, H = encoder_output.shape
    N = B * S
    NBINS = pitch_table.shape[0]
    enc = encoder_output.reshape(N, H)
    pv = pitch_target.reshape(N)
    ev = energy_target.reshape(N)
    ctab = jnp.concatenate([pitch_table, energy_table], axis=0)
    bnds = jnp.concatenate([
        jnp.linspace(50.0, 400.0, NBINS),
        jnp.linspace(0.0, 1.0, NBINS),
    ])
    out = _sc_call(N, H, NBINS)(enc, pv, ev, ctab, bnds)
    return out.reshape(B, S, H)


# speedup vs baseline: 8.1552x; 1.3547x over previous
import functools

import jax
import jax.numpy as jnp
from jax import lax
from jax.experimental import pallas as pl
from jax.experimental.pallas import tpu as pltpu
from jax.experimental.pallas import tpu_sc as plsc

NC, NS, L = 2, 16, 16
NW = NC * NS
CW = 128
T = 128


def _sc_call(N, H, NBINS):
    SPT = N // NS
    CG = H // CW
    TS = NW // CG
    NPS = N // TS
    CHUNKS = NPS // T
    R = 2 * NBINS
    CH = CW // L

    mesh = plsc.VectorSubcoreMesh(core_axis_name="c", subcore_axis_name="s")

    @functools.partial(
        pl.kernel,
        out_type=jax.ShapeDtypeStruct((N, H), jnp.float32),
        mesh=mesh,
        compiler_params=pltpu.CompilerParams(needs_layout_passes=False),
        scratch_types=[
            pltpu.VMEM((R, CW), jnp.float32),
            pltpu.VMEM((2, T, CW), jnp.float32),
            pltpu.VMEM((2, T), jnp.int32),
            pltpu.VMEM((2, T), jnp.int32),
            pltpu.VMEM((SPT,), jnp.float32),
            pltpu.VMEM((SPT,), jnp.float32),
            pltpu.VMEM((SPT,), jnp.int32),
            pltpu.VMEM((SPT,), jnp.int32),
            pltpu.VMEM((2 * NBINS,), jnp.float32),
            pltpu.VMEM_SHARED((N,), jnp.int32),
            pltpu.VMEM_SHARED((N,), jnp.int32),
            pltpu.SemaphoreType.DMA,
            pltpu.SemaphoreType.DMA,
            pltpu.SemaphoreType.DMA,
            pltpu.SemaphoreType.DMA,
            pltpu.SemaphoreType.DMA,
            pltpu.SemaphoreType.DMA,
        ],
    )
    def body(enc_hbm, pv_hbm, ev_hbm, ctab_hbm, bnd_hbm, out_hbm,
             tab, out_b, pb_b, eb_b, pvals, evals, pidx, eidx, bnds,
             pidx_sh, eidx_sh,
             se0, se1, si0, si1, sw0, sw1):
        cid = lax.axis_index("c")
        sid = lax.axis_index("s")
        wid = cid * NS + sid
        gcol = (wid % CG) * CW
        tok0 = (wid // CG) * NPS
        se, si, sw = (se0, se1), (si0, si1), (sw0, sw1)

        pltpu.sync_copy(bnd_hbm, bnds)
        qbase = sid * SPT
        pltpu.sync_copy(pv_hbm.at[pl.ds(qbase, SPT)], pvals)
        pltpu.sync_copy(ev_hbm.at[pl.ds(qbase, SPT)], evals)

        @plsc.parallel_loop(0, SPT // L)
        def _search(j):
            sl = pl.ds(j * L, L)
            for vals_ref, idx_ref, base_bin in ((pvals, pidx, 0),
                                                (evals, eidx, NBINS)):
                v = vals_ref[sl]
                curr = jnp.zeros((L,), jnp.int32)
                step = NBINS // 2
                while step >= 1:
                    probe = plsc.load_gather(bnds, [curr + (base_bin + step - 1)])
                    curr = jnp.where(probe < v, curr + step, curr)
                    step //= 2
                idx_ref[sl] = curr + base_bin

        pltpu.sync_copy(pidx, pidx_sh.at[pl.ds(qbase, SPT)])
        pltpu.sync_copy(eidx, eidx_sh.at[pl.ds(qbase, SPT)])

        pltpu.sync_copy(ctab_hbm.at[:, pl.ds(gcol, CW)], tab)
        plsc.subcore_barrier()

        def issue(c, p):
            base = tok0 + c * T
            pltpu.async_copy(enc_hbm.at[pl.ds(base, T), pl.ds(gcol, CW)],
                             out_b.at[p], se[p])
            pltpu.async_copy(pidx_sh.at[pl.ds(base, T)], pb_b.at[p], si[p])
            pltpu.async_copy(eidx_sh.at[pl.ds(base, T)], eb_b.at[p], si[p])

        def wait_wb(p):
            pltpu.make_async_copy(out_b.at[p],
                                  out_hbm.at[pl.ds(tok0, T), pl.ds(gcol, CW)],
                                  sw[p]).wait()

        def finish(c, p):
            base = tok0 + c * T
            pltpu.make_async_copy(enc_hbm.at[pl.ds(base, T), pl.ds(gcol, CW)],
                                  out_b.at[p], se[p]).wait()
            pltpu.make_async_copy(pidx_sh.at[pl.ds(base, T)], pb_b.at[p],
                                  si[p]).wait()
            pltpu.make_async_copy(eidx_sh.at[pl.ds(base, T)], eb_b.at[p],
                                  si[p]).wait()

            @plsc.parallel_loop(0, T // L)
            def _row(j):
                t0 = j * L
                rpv = pb_b[p, pl.ds(t0, L)]
                rev = eb_b[p, pl.ds(t0, L)]
                for k in range(L):
                    for h in range(CH):
                        sl = pl.ds(h * L, L)
                        plsc.addupdate(out_b.at[p, t0 + k, sl],
                                       tab[rpv[k], sl] + tab[rev[k], sl])

            pltpu.async_copy(out_b.at[p],
                             out_hbm.at[pl.ds(base, T), pl.ds(gcol, CW)],
                             sw[p])

        issue(0, 0)

        @pl.loop(0, CHUNKS, step=2)
        def _main(cc):
            @pl.when(cc > 0)
            def _():
                wait_wb(1)

            issue(cc + 1, 1)
            finish(cc, 0)

            @pl.when(cc + 2 < CHUNKS)
            def _():
                wait_wb(0)
                issue(cc + 2, 0)

            finish(cc + 1, 1)

        wait_wb(0)
        wait_wb(1)

    return body


def kernel(encoder_output, pitch_target, energy_target, pitch_table, energy_table):
    B, S, H = encoder_output.shape
    N = B * S
    NBINS = pitch_table.shape[0]
    enc = encoder_output.reshape(N, H)
    pv = pitch_target.reshape(N)
    ev = energy_target.reshape(N)
    ctab = jnp.concatenate([pitch_table, energy_table], axis=0)
    bnds = jnp.concatenate([
        jnp.linspace(50.0, 400.0, NBINS),
        jnp.linspace(0.0, 1.0, NBINS),
    ])
    out = _sc_call(N, H, NBINS)(enc, pv, ev, ctab, bnds)
    return out.reshape(B, S, H)


# speedup vs baseline: 10.5840x; 1.2978x over previous
import functools

import jax
import jax.numpy as jnp
from jax import lax
from jax.experimental import pallas as pl
from jax.experimental.pallas import tpu as pltpu
from jax.experimental.pallas import tpu_sc as plsc

NC, NS, L = 2, 16, 16
NW = NC * NS
CW = 128
T = 128


def _sc_call(N, H, NBINS):
    SPT = N // NS
    CG = H // CW
    TS = NW // CG
    NPS = N // TS
    CHUNKS = NPS // T
    R = 2 * NBINS
    CH = CW // L

    mesh = plsc.VectorSubcoreMesh(core_axis_name="c", subcore_axis_name="s")

    @functools.partial(
        pl.kernel,
        out_type=jax.ShapeDtypeStruct((N, H), jnp.float32),
        mesh=mesh,
        compiler_params=pltpu.CompilerParams(needs_layout_passes=False),
        scratch_types=[
            pltpu.VMEM((R, CW), jnp.bfloat16),
            pltpu.VMEM((2, T, CW), jnp.float32),
            pltpu.VMEM((2, T), jnp.int32),
            pltpu.VMEM((2, T), jnp.int32),
            pltpu.VMEM((SPT,), jnp.float32),
            pltpu.VMEM((SPT,), jnp.float32),
            pltpu.VMEM((SPT,), jnp.int32),
            pltpu.VMEM((SPT,), jnp.int32),
            pltpu.VMEM((2 * NBINS,), jnp.float32),
            pltpu.VMEM_SHARED((N,), jnp.int32),
            pltpu.VMEM_SHARED((N,), jnp.int32),
            pltpu.SemaphoreType.DMA,
            pltpu.SemaphoreType.DMA,
            pltpu.SemaphoreType.DMA,
            pltpu.SemaphoreType.DMA,
            pltpu.SemaphoreType.DMA,
            pltpu.SemaphoreType.DMA,
        ],
    )
    def body(enc_hbm, pv_hbm, ev_hbm, ctab_hbm, bnd_hbm, out_hbm,
             tab, out_b, pb_b, eb_b, pvals, evals, pidx, eidx, bnds,
             pidx_sh, eidx_sh,
             se0, se1, si0, si1, sw0, sw1):
        cid = lax.axis_index("c")
        sid = lax.axis_index("s")
        wid = cid * NS + sid
        gcol = (wid % CG) * CW
        tok0 = (wid // CG) * NPS
        se, si, sw = (se0, se1), (si0, si1), (sw0, sw1)

        pltpu.sync_copy(bnd_hbm, bnds)
        qbase = sid * SPT
        pltpu.sync_copy(pv_hbm.at[pl.ds(qbase, SPT)], pvals)
        pltpu.sync_copy(ev_hbm.at[pl.ds(qbase, SPT)], evals)

        @plsc.parallel_loop(0, SPT // L)
        def _search(j):
            sl = pl.ds(j * L, L)
            for vals_ref, idx_ref, base_bin in ((pvals, pidx, 0),
                                                (evals, eidx, NBINS)):
                v = vals_ref[sl]
                curr = jnp.zeros((L,), jnp.int32)
                step = NBINS // 2
                while step >= 1:
                    probe = plsc.load_gather(bnds, [curr + (base_bin + step - 1)])
                    curr = jnp.where(probe < v, curr + step, curr)
                    step //= 2
                idx_ref[sl] = curr + base_bin

        pltpu.sync_copy(pidx, pidx_sh.at[pl.ds(qbase, SPT)])
        pltpu.sync_copy(eidx, eidx_sh.at[pl.ds(qbase, SPT)])

        pltpu.sync_copy(ctab_hbm.at[wid % CG], tab)
        plsc.subcore_barrier()

        def issue(c, p):
            base = tok0 + c * T
            pltpu.async_copy(enc_hbm.at[pl.ds(base, T), pl.ds(gcol, CW)],
                             out_b.at[p], se[p])
            pltpu.async_copy(pidx_sh.at[pl.ds(base, T)], pb_b.at[p], si[p])
            pltpu.async_copy(eidx_sh.at[pl.ds(base, T)], eb_b.at[p], si[p])

        def wait_wb(p):
            pltpu.make_async_copy(out_b.at[p],
                                  out_hbm.at[pl.ds(tok0, T), pl.ds(gcol, CW)],
                                  sw[p]).wait()

        def finish(c, p):
            base = tok0 + c * T
            pltpu.make_async_copy(enc_hbm.at[pl.ds(base, T), pl.ds(gcol, CW)],
                                  out_b.at[p], se[p]).wait()
            pltpu.make_async_copy(pidx_sh.at[pl.ds(base, T)], pb_b.at[p],
                                  si[p]).wait()
            pltpu.make_async_copy(eidx_sh.at[pl.ds(base, T)], eb_b.at[p],
                                  si[p]).wait()

            @plsc.parallel_loop(0, T // L)
            def _row(j):
                t0 = j * L
                rpv = pb_b[p, pl.ds(t0, L)]
                rev = eb_b[p, pl.ds(t0, L)]
                for k in range(L):
                    for h2 in range(CW // (2 * L)):
                        sl = pl.ds(h2 * 2 * L, 2 * L)
                        pa, pb = plsc.unpack(
                            tab[rpv[k], sl], format=plsc.PackFormat.INTERLEAVED)
                        ea, eb = plsc.unpack(
                            tab[rev[k], sl], format=plsc.PackFormat.INTERLEAVED)
                        plsc.addupdate(
                            out_b.at[p, t0 + k, pl.ds(h2 * 2 * L, L)], pa + ea)
                        plsc.addupdate(
                            out_b.at[p, t0 + k, pl.ds(h2 * 2 * L + L, L)],
                            pb + eb)

            pltpu.async_copy(out_b.at[p],
                             out_hbm.at[pl.ds(base, T), pl.ds(gcol, CW)],
                             sw[p])

        issue(0, 0)

        @pl.loop(0, CHUNKS, step=2)
        def _main(cc):
            @pl.when(cc > 0)
            def _():
                wait_wb(1)

            issue(cc + 1, 1)
            finish(cc, 0)

            @pl.when(cc + 2 < CHUNKS)
            def _():
                wait_wb(0)
                issue(cc + 2, 0)

            finish(cc + 1, 1)

        wait_wb(0)
        wait_wb(1)

    return body


def kernel(encoder_output, pitch_target, energy_target, pitch_table, energy_table):
    B, S, H = encoder_output.shape
    N = B * S
    NBINS = pitch_table.shape[0]
    enc = encoder_output.reshape(N, H)
    pv = pitch_target.reshape(N)
    ev = energy_target.reshape(N)
    ctab = jnp.concatenate([pitch_table, energy_table], axis=0)
    R = 2 * NBINS
    CG = H // CW
    ctab = (ctab.astype(jnp.bfloat16)
            .reshape(R, CG, CW // 32, 2, 16)
            .transpose(1, 0, 2, 4, 3)
            .reshape(CG, R, CW))
    bnds = jnp.concatenate([
        jnp.linspace(50.0, 400.0, NBINS),
        jnp.linspace(0.0, 1.0, NBINS),
    ])
    out = _sc_call(N, H, NBINS)(enc, pv, ev, ctab, bnds)
    return out.reshape(B, S, H)
